# baseline (speedup 1.0000x reference)
import functools

import jax
import jax.numpy as jnp
from jax import lax
from jax.experimental import pallas as pl
from jax.experimental.pallas import tpu as pltpu
from jax.experimental.pallas import tpu_sc as plsc

_NC = 2
_NS = 16
_NW = _NC * _NS

_BR = 256
_GCHUNK = 128


def _deg_zproj_body(adj_ref, zd_ref, w2t_ref, deg_ref, zproj_ref, max_degree):
    adj = adj_ref[...]
    deg = jnp.sum(adj, axis=1)
    deg_ref[0, 0, :] = jnp.minimum(deg, max_degree)

    @pl.when(pl.program_id(0) == 0)
    def _():
        zproj_ref[...] = 0.3 * jnp.dot(
            zd_ref[...], w2t_ref[...], preferred_element_type=jnp.float32
        )


def _out_body(x_ref, w1t_ref, b_ref, g_ref, out_ref):
    out_ref[...] = (
        jnp.dot(0.7 * x_ref[...], w1t_ref[...], preferred_element_type=jnp.float32)
        + b_ref[...]
        + g_ref[...]
    )


def _make_sc_gather(n, d, rows_per_w):
    chunks = rows_per_w // _GCHUNK
    mesh = plsc.VectorSubcoreMesh(core_axis_name="c", subcore_axis_name="s")

    @functools.partial(
        pl.kernel,
        out_type=jax.ShapeDtypeStruct((n, d), jnp.float32),
        mesh=mesh,
        scratch_types=[
            pltpu.VMEM((chunks, _GCHUNK), jnp.int32),
            pltpu.VMEM((chunks, _GCHUNK, d), jnp.float32),
            pltpu.SemaphoreType.DMA,
        ],
    )
    def sc_gather(deg_hbm, zproj_hbm, out_hbm, idx_v, rows_v, sem):
        wid = lax.axis_index("s") * _NC + lax.axis_index("c")
        row0 = wid * rows_per_w
        pltpu.sync_copy(deg_hbm.at[pl.ds(wid * chunks, chunks)], idx_v)
        copies = [
            pltpu.async_copy(zproj_hbm.at[idx_v.at[j]], rows_v.at[j], sem)
            for j in range(chunks)
        ]
        for c in copies:
            c.wait()
        for j in range(chunks):
            pltpu.sync_copy(
                rows_v.at[j], out_hbm.at[pl.ds(row0 + j * _GCHUNK, _GCHUNK)]
            )

    return sc_gather


def kernel(x, adj, z_degree, W, b):
    n, d = x.shape
    max_degree = z_degree.shape[0] - 1
    w1t = W[:, :d].T
    w2t = W[:, d:].T
    b_row = b.reshape(1, d)

    nblocks = n // _BR
    deg3, zproj = pl.pallas_call(
        functools.partial(_deg_zproj_body, max_degree=max_degree),
        grid=(nblocks,),
        in_specs=[
            pl.BlockSpec((_BR, n), lambda i: (i, 0)),
            pl.BlockSpec(z_degree.shape, lambda i: (0, 0)),
            pl.BlockSpec((d, d), lambda i: (0, 0)),
        ],
        out_specs=[
            pl.BlockSpec((1, 1, _BR), lambda i: (i, 0, 0)),
            pl.BlockSpec((z_degree.shape[0], d), lambda i: (0, 0)),
        ],
        out_shape=[
            jax.ShapeDtypeStruct((nblocks, 1, _BR), jnp.int32),
            jax.ShapeDtypeStruct((z_degree.shape[0], d), jnp.float32),
        ],
    )(adj, z_degree, w2t)

    deg2 = deg3.reshape(n // _GCHUNK, _GCHUNK)
    rows_per_w = n // _NW
    gathered = _make_sc_gather(n, d, rows_per_w)(deg2, zproj)

    bx = 512
    out = pl.pallas_call(
        _out_body,
        grid=(n // bx,),
        in_specs=[
            pl.BlockSpec((bx, d), lambda i: (i, 0)),
            pl.BlockSpec((d, d), lambda i: (0, 0)),
            pl.BlockSpec((1, d), lambda i: (0, 0)),
            pl.BlockSpec((bx, d), lambda i: (i, 0)),
        ],
        out_specs=pl.BlockSpec((bx, d), lambda i: (i, 0)),
        out_shape=jax.ShapeDtypeStruct((n, d), jnp.float32),
    )(x, w1t, b_row, gathered)
    return out


# speedup vs baseline: 3.9298x; 3.9298x over previous
import functools

import jax
import jax.numpy as jnp
from jax import lax
from jax.experimental import pallas as pl
from jax.experimental.pallas import tpu as pltpu
from jax.experimental.pallas import tpu_sc as plsc

_NC = 2
_NS = 16
_NW = _NC * _NS
_LANES = 16

_BR = 256
_GCHUNK = 128


def _k1_body(adj_ref, x_ref, w1t_ref, b_ref, zd_ref, w2t_ref,
             deg_ref, part_ref, zproj_ref, max_degree):
    adj = adj_ref[...]
    deg = jnp.sum(adj, axis=1)
    deg_ref[0, 0, :] = jnp.minimum(deg, max_degree)
    part_ref[...] = (
        jnp.dot(0.7 * x_ref[...], w1t_ref[...], preferred_element_type=jnp.float32)
        + b_ref[...]
    )

    @pl.when(pl.program_id(0) == 0)
    def _():
        zproj_ref[...] = 0.3 * jnp.dot(
            zd_ref[...], w2t_ref[...], preferred_element_type=jnp.float32
        )


def _make_sc_finisher(n, d, v):
    rows_per_w = n // _NW
    chunks = rows_per_w // _GCHUNK
    mesh = plsc.VectorSubcoreMesh(core_axis_name="c", subcore_axis_name="s")

    @functools.partial(
        pl.kernel,
        out_type=jax.ShapeDtypeStruct((n // _GCHUNK, _GCHUNK, d), jnp.float32),
        mesh=mesh,
        scratch_types=[
            pltpu.VMEM_SHARED((v, d), jnp.float32),
            pltpu.VMEM((chunks, _GCHUNK), jnp.int32),
            pltpu.VMEM((chunks, _GCHUNK, d), jnp.float32),
            pltpu.VMEM((chunks, _GCHUNK, d), jnp.float32),
            pltpu.SemaphoreType.DMA,
            pltpu.SemaphoreType.DMA,
        ],
    )
    def sc_finish(deg_hbm, part_hbm, zproj_hbm, out_hbm,
                  tab_sh, idx_v, rows_v, part_v, sem_g, sem_p):
        c = lax.axis_index("c")
        s = lax.axis_index("s")
        wid = s * _NC + c

        @pl.when(s == 0)
        def _():
            pltpu.sync_copy(zproj_hbm, tab_sh)
        plsc.subcore_barrier()

        pltpu.sync_copy(deg_hbm.at[pl.ds(wid * chunks, chunks)], idx_v)
        pcopy = pltpu.async_copy(
            part_hbm.at[pl.ds(wid * chunks, chunks)], part_v, sem_p
        )
        gcopies = [
            pltpu.async_copy(tab_sh.at[idx_v.at[j]], rows_v.at[j], sem_g)
            for j in range(chunks)
        ]
        pcopy.wait()
        for g in gcopies:
            g.wait()

        for j in range(chunks):
            def body(i, _, j=j):
                for t in range(d // _LANES):
                    sl = pl.ds(t * _LANES, _LANES)
                    rows_v[j, i, sl] = rows_v[j, i, sl] + part_v[j, i, sl]
                return 0
            lax.fori_loop(0, _GCHUNK, body, 0)

        pltpu.sync_copy(rows_v, out_hbm.at[pl.ds(wid * chunks, chunks)])

    return sc_finish


def kernel(x, adj, z_degree, W, b):
    n, d = x.shape
    v = z_degree.shape[0]
    max_degree = v - 1
    w1t = W[:, :d].T
    w2t = W[:, d:].T
    b_row = b.reshape(1, d)

    nblocks = n // _BR
    deg3, part, zproj = pl.pallas_call(
        functools.partial(_k1_body, max_degree=max_degree),
        grid=(nblocks,),
        in_specs=[
            pl.BlockSpec((_BR, n), lambda i: (i, 0)),
            pl.BlockSpec((_BR, d), lambda i: (i, 0)),
            pl.BlockSpec((d, d), lambda i: (0, 0)),
            pl.BlockSpec((1, d), lambda i: (0, 0)),
            pl.BlockSpec((v, d), lambda i: (0, 0)),
            pl.BlockSpec((d, d), lambda i: (0, 0)),
        ],
        out_specs=[
            pl.BlockSpec((1, 1, _BR), lambda i: (i, 0, 0)),
            pl.BlockSpec((_BR, d), lambda i: (i, 0)),
            pl.BlockSpec((v, d), lambda i: (0, 0)),
        ],
        out_shape=[
            jax.ShapeDtypeStruct((nblocks, 1, _BR), jnp.int32),
            jax.ShapeDtypeStruct((n, d), jnp.float32),
            jax.ShapeDtypeStruct((v, d), jnp.float32),
        ],
    )(adj, x, w1t, b_row, z_degree, w2t)

    deg2 = deg3.reshape(n // _GCHUNK, _GCHUNK)
    part3 = part.reshape(n // _GCHUNK, _GCHUNK, d)
    out3 = _make_sc_finisher(n, d, v)(deg2, part3, zproj)
    return out3.reshape(n, d)


# speedup vs baseline: 4.0175x; 1.0223x over previous
import functools

import jax
import jax.numpy as jnp
from jax import lax
from jax.experimental import pallas as pl
from jax.experimental.pallas import tpu as pltpu
from jax.experimental.pallas import tpu_sc as plsc

_NC = 2
_NS = 16
_NW = _NC * _NS
_LANES = 16

_BR = 256
_GCHUNK = 128


_DNUMS = (((1,), (1,)), ((), ()))


def _k1_body(adj_ref, x_ref, w1_ref, b_ref, zd_ref, w2_ref,
             deg_ref, part_ref, zproj_ref, max_degree):
    adj = adj_ref[...]
    deg = jnp.sum(adj, axis=1)
    deg_ref[0, 0, :] = jnp.minimum(deg, max_degree)
    part_ref[...] = (
        lax.dot_general(0.7 * x_ref[...], w1_ref[...], _DNUMS,
                        preferred_element_type=jnp.float32)
        + b_ref[...]
    )

    @pl.when(pl.program_id(0) == 0)
    def _():
        zproj_ref[...] = 0.3 * lax.dot_general(
            zd_ref[...], w2_ref[...], _DNUMS, preferred_element_type=jnp.float32
        )


def _make_sc_finisher(n, d, v):
    rows_per_w = n // _NW
    chunks = rows_per_w // _GCHUNK
    mesh = plsc.VectorSubcoreMesh(core_axis_name="c", subcore_axis_name="s")

    @functools.partial(
        pl.kernel,
        out_type=jax.ShapeDtypeStruct((n // _GCHUNK, _GCHUNK, d), jnp.float32),
        mesh=mesh,
        scratch_types=[
            pltpu.VMEM_SHARED((v, d), jnp.float32),
            pltpu.VMEM((chunks, _GCHUNK), jnp.int32),
            pltpu.VMEM((chunks, _GCHUNK, d), jnp.float32),
            pltpu.VMEM((chunks, _GCHUNK, d), jnp.float32),
            pltpu.SemaphoreType.DMA,
            pltpu.SemaphoreType.DMA,
        ],
    )
    def sc_finish(deg_hbm, part_hbm, zproj_hbm, out_hbm,
                  tab_sh, idx_v, rows_v, part_v, sem_g, sem_p):
        c = lax.axis_index("c")
        s = lax.axis_index("s")
        wid = s * _NC + c

        @pl.when(s == 0)
        def _():
            pltpu.sync_copy(zproj_hbm, tab_sh)
        plsc.subcore_barrier()

        pltpu.sync_copy(deg_hbm.at[pl.ds(wid * chunks, chunks)], idx_v)
        pcopy = pltpu.async_copy(
            part_hbm.at[pl.ds(wid * chunks, chunks)], part_v, sem_p
        )
        gcopies = [
            pltpu.async_copy(tab_sh.at[idx_v.at[j]], rows_v.at[j], sem_g)
            for j in range(chunks)
        ]
        pcopy.wait()
        for g in gcopies:
            g.wait()

        for j in range(chunks):
            def body(i, _, j=j):
                for t in range(d // _LANES):
                    sl = pl.ds(t * _LANES, _LANES)
                    rows_v[j, i, sl] = rows_v[j, i, sl] + part_v[j, i, sl]
                return 0
            lax.fori_loop(0, _GCHUNK, body, 0)

        pltpu.sync_copy(rows_v, out_hbm.at[pl.ds(wid * chunks, chunks)])

    return sc_finish


def kernel(x, adj, z_degree, W, b):
    n, d = x.shape
    v = z_degree.shape[0]
    max_degree = v - 1
    b_row = b.reshape(1, d)

    nblocks = n // _BR
    deg3, part, zproj = pl.pallas_call(
        functools.partial(_k1_body, max_degree=max_degree),
        grid=(nblocks,),
        in_specs=[
            pl.BlockSpec((_BR, n), lambda i: (i, 0)),
            pl.BlockSpec((_BR, d), lambda i: (i, 0)),
            pl.BlockSpec((d, d), lambda i: (0, 0)),
            pl.BlockSpec((1, d), lambda i: (0, 0)),
            pl.BlockSpec((v, d), lambda i: (0, 0)),
            pl.BlockSpec((d, d), lambda i: (0, 1)),
        ],
        out_specs=[
            pl.BlockSpec((1, 1, _BR), lambda i: (i, 0, 0)),
            pl.BlockSpec((_BR, d), lambda i: (i, 0)),
            pl.BlockSpec((v, d), lambda i: (0, 0)),
        ],
        out_shape=[
            jax.ShapeDtypeStruct((nblocks, 1, _BR), jnp.int32),
            jax.ShapeDtypeStruct((n, d), jnp.float32),
            jax.ShapeDtypeStruct((v, d), jnp.float32),
        ],
    )(adj, x, W, b_row, z_degree, W)

    deg2 = deg3.reshape(n // _GCHUNK, _GCHUNK)
    part3 = part.reshape(n // _GCHUNK, _GCHUNK, d)
    out3 = _make_sc_finisher(n, d, v)(deg2, part3, zproj)
    return out3.reshape(n, d)
